# baseline (speedup 1.0000x reference)
import functools
import numpy as np
import jax
from jax import lax
import jax.numpy as jnp
from jax.experimental import pallas as pl
from jax.experimental.pallas import tpu as pltpu
from jax.experimental.pallas import tpu_sc as plsc

_B = 128
_D = 3 * 224 * 224
_ALPHA = 0.9
_BETA = 1.0 - _ALPHA

_NC, _NS = 2, 16
_NW = _NC * _NS
_ROWS_PER_W = _B // _NW
_CH = 84
_C = _D // _CH
_GRP = 8
_NGRP = _ROWS_PER_W * _CH // _GRP
_CPW = _ROWS_PER_W * _CH


_PERM = np.asarray([
    98, 105, 103, 43, 22, 94, 86, 125, 49, 0, 45, 108, 56, 121, 62, 109,
    3, 77, 9, 64, 5, 52, 50, 37, 78, 95, 30, 117, 127, 71, 53, 34,
    83, 18, 14, 116, 46, 1, 74, 124, 58, 92, 51, 81, 107, 48, 100, 42,
    106, 8, 69, 101, 90, 110, 66, 65, 21, 17, 67, 4, 32, 102, 27, 33,
    75, 89, 70, 123, 63, 104, 13, 39, 73, 85, 79, 120, 91, 41, 115, 6,
    59, 2, 57, 35, 99, 19, 40, 72, 118, 54, 80, 31, 126, 26, 97, 36,
    38, 25, 47, 61, 96, 15, 28, 68, 60, 82, 112, 55, 44, 119, 11, 114,
    10, 122, 76, 93, 84, 87, 16, 12, 88, 23, 29, 24, 7, 113, 111, 20,
], dtype=np.int32)


def _build_bidx():
    w = np.arange(_NW)[:, None]
    q = np.arange(_CPW)[None, :]
    rows = _PERM[4 * w + q // _CH]
    return (rows * _CH + q % _CH).astype(np.int32)


_BIDX = _build_bidx()


def _axpy_group(a_ref, b_ref, o_ref):
    @plsc.parallel_loop(0, _C // 112)
    def it(t):
        for r in range(_GRP):
            for u in range(7):
                sl = pl.ds(t * 112 + u * 16, 16)
                o_ref[r, sl] = _ALPHA * a_ref[r, sl] + _BETA * b_ref[r, sl]


def _sc_body(x_hbm, bidx_hbm, out_hbm, idx_v,
             a0, a1, b0, b1, o0, o1,
             sa0, sa1, sb0, sb1, so0, so1):
    wid = lax.axis_index("s") * _NC + lax.axis_index("c")
    abase = wid * _CPW
    abufs, bbufs, obufs = (a0, a1), (b0, b1), (o0, o1)
    sas, sbs, sos = (sa0, sa1), (sb0, sb1), (so0, so1)

    pltpu.sync_copy(bidx_hbm.at[wid], idx_v)

    def a_src(s):
        return x_hbm.at[pl.ds(abase + _GRP * s, _GRP)]

    def b_src(s):
        return x_hbm.at[idx_v.at[pl.ds(_GRP * s, _GRP)]]

    def o_dst(s):
        return out_hbm.at[pl.ds(abase + _GRP * s, _GRP)]

    for j in range(2):
        pltpu.make_async_copy(a_src(j), abufs[j], sas[j]).start()
        pltpu.make_async_copy(b_src(j), bbufs[j], sbs[j]).start()

    def step(g, c):
        for j in range(2):
            s = g * 2 + j
            pltpu.make_async_copy(a_src(s), abufs[j], sas[j]).wait()
            pltpu.make_async_copy(b_src(s), bbufs[j], sbs[j]).wait()

            @pl.when(s >= 2)
            def _():
                pltpu.make_async_copy(obufs[j], o_dst(s - 2), sos[j]).wait()

            _axpy_group(abufs[j], bbufs[j], obufs[j])
            pltpu.make_async_copy(obufs[j], o_dst(s), sos[j]).start()

            @pl.when(s < _NGRP - 2)
            def _():
                pltpu.make_async_copy(a_src(s + 2), abufs[j], sas[j]).start()
                pltpu.make_async_copy(b_src(s + 2), bbufs[j], sbs[j]).start()
        return c

    lax.fori_loop(0, _NGRP // 2, step, 0)

    for j in range(2):
        pltpu.make_async_copy(obufs[j], o_dst(_NGRP - 2 + j), sos[j]).wait()


@functools.partial(
    pl.kernel,
    out_type=jax.ShapeDtypeStruct((_B * _CH, _C), jnp.float32),
    mesh=plsc.VectorSubcoreMesh(core_axis_name="c", subcore_axis_name="s"),
    scratch_types=[
        pltpu.VMEM((_CPW,), jnp.int32),
        pltpu.VMEM((_GRP, _C), jnp.float32),
        pltpu.VMEM((_GRP, _C), jnp.float32),
        pltpu.VMEM((_GRP, _C), jnp.float32),
        pltpu.VMEM((_GRP, _C), jnp.float32),
        pltpu.VMEM((_GRP, _C), jnp.float32),
        pltpu.VMEM((_GRP, _C), jnp.float32),
        pltpu.SemaphoreType.DMA,
        pltpu.SemaphoreType.DMA,
        pltpu.SemaphoreType.DMA,
        pltpu.SemaphoreType.DMA,
        pltpu.SemaphoreType.DMA,
        pltpu.SemaphoreType.DMA,
    ],
)
def _mixup_sc(x_hbm, bidx_hbm, out_hbm, *scratch):
    _sc_body(x_hbm, bidx_hbm, out_hbm, *scratch)


def kernel(x):
    x2 = x.reshape(_B * _CH, _C)
    out2 = _mixup_sc(x2, jnp.asarray(_BIDX))
    return out2.reshape(x.shape)


# speedup vs baseline: 1.6808x; 1.6808x over previous
import functools
import numpy as np
import jax
from jax import lax
import jax.numpy as jnp
from jax.experimental import pallas as pl
from jax.experimental.pallas import tpu as pltpu
from jax.experimental.pallas import tpu_sc as plsc

_B = 128
_CC, _HH, _WW = 3, 224, 224
_ALPHA = 0.9
_BETA = 1.0 - _ALPHA

_NC, _NS, _L = 2, 16, 16
_NW = _NC * _NS
_ROWS_PER_W = _B // _NW
_HB = 56
_NHB = _HH // _HB
_STEPS = _ROWS_PER_W * _CC * _NHB


_PERM = np.asarray([
    98, 105, 103, 43, 22, 94, 86, 125, 49, 0, 45, 108, 56, 121, 62, 109,
    3, 77, 9, 64, 5, 52, 50, 37, 78, 95, 30, 117, 127, 71, 53, 34,
    83, 18, 14, 116, 46, 1, 74, 124, 58, 92, 51, 81, 107, 48, 100, 42,
    106, 8, 69, 101, 90, 110, 66, 65, 21, 17, 67, 4, 32, 102, 27, 33,
    75, 89, 70, 123, 63, 104, 13, 39, 73, 85, 79, 120, 91, 41, 115, 6,
    59, 2, 57, 35, 99, 19, 40, 72, 118, 54, 80, 31, 126, 26, 97, 36,
    38, 25, 47, 61, 96, 15, 28, 68, 60, 82, 112, 55, 44, 119, 11, 114,
    10, 122, 76, 93, 84, 87, 16, 12, 88, 23, 29, 24, 7, 113, 111, 20,
], dtype=np.int32)


def _axpy_chunk(a_ref, b_ref, o_ref):
    @plsc.parallel_loop(0, _HB)
    def it(t):
        for u in range(_WW // _L):
            sl = pl.ds(u * _L, _L)
            o_ref[t, sl] = _ALPHA * a_ref[t, sl] + _BETA * b_ref[t, sl]


def _perm_lookup(row):
    bi = jnp.int32(0)
    for k in range(_B):
        bi = jnp.where(row == k, jnp.int32(int(_PERM[k])), bi)
    return bi


def _sc_body(x_hbm, out_hbm,
             a0, a1, b0, b1, o0, o1,
             sa0, sa1, sb0, sb1, so0, so1):
    wid = lax.axis_index("s") * _NC + lax.axis_index("c")
    abufs, bbufs, obufs = (a0, a1), (b0, b1), (o0, o1)
    sas, sbs, sos = (sa0, sa1), (sb0, sb1), (so0, so1)

    bis = [_perm_lookup(4 * wid + r) for r in range(_ROWS_PER_W)]

    def split(s):
        r = s // (_CC * _NHB)
        c = (s % (_CC * _NHB)) // _NHB
        hb = s % _NHB
        return r, c, hb

    def a_src(s):
        r, c, hb = split(s)
        return x_hbm.at[4 * wid + r, c, pl.ds(_HB * hb, _HB)]

    def b_src(s):
        r, c, hb = split(s)
        bi = jnp.where(r == 0, bis[0],
                       jnp.where(r == 1, bis[1],
                                 jnp.where(r == 2, bis[2], bis[3])))
        return x_hbm.at[bi, c, pl.ds(_HB * hb, _HB)]

    def o_dst(s):
        r, c, hb = split(s)
        return out_hbm.at[4 * wid + r, c, pl.ds(_HB * hb, _HB)]

    for j in range(2):
        pltpu.make_async_copy(a_src(j), abufs[j], sas[j]).start()
        pltpu.make_async_copy(b_src(j), bbufs[j], sbs[j]).start()

    def step(g, carry):
        for j in range(2):
            s = g * 2 + j
            pltpu.make_async_copy(a_src(s), abufs[j], sas[j]).wait()
            pltpu.make_async_copy(b_src(s), bbufs[j], sbs[j]).wait()

            @pl.when(s >= 2)
            def _():
                pltpu.make_async_copy(obufs[j], o_dst(s - 2), sos[j]).wait()

            _axpy_chunk(abufs[j], bbufs[j], obufs[j])
            pltpu.make_async_copy(obufs[j], o_dst(s), sos[j]).start()

            @pl.when(s < _STEPS - 2)
            def _():
                pltpu.make_async_copy(a_src(s + 2), abufs[j], sas[j]).start()
                pltpu.make_async_copy(b_src(s + 2), bbufs[j], sbs[j]).start()
        return carry

    lax.fori_loop(0, _STEPS // 2, step, 0)

    for j in range(2):
        pltpu.make_async_copy(obufs[j], o_dst(_STEPS - 2 + j), sos[j]).wait()


@functools.partial(
    pl.kernel,
    out_type=jax.ShapeDtypeStruct((_B, _CC, _HH, _WW), jnp.float32),
    mesh=plsc.VectorSubcoreMesh(core_axis_name="c", subcore_axis_name="s"),
    scratch_types=[
        pltpu.VMEM((_HB, _WW), jnp.float32),
        pltpu.VMEM((_HB, _WW), jnp.float32),
        pltpu.VMEM((_HB, _WW), jnp.float32),
        pltpu.VMEM((_HB, _WW), jnp.float32),
        pltpu.VMEM((_HB, _WW), jnp.float32),
        pltpu.VMEM((_HB, _WW), jnp.float32),
        pltpu.SemaphoreType.DMA,
        pltpu.SemaphoreType.DMA,
        pltpu.SemaphoreType.DMA,
        pltpu.SemaphoreType.DMA,
        pltpu.SemaphoreType.DMA,
        pltpu.SemaphoreType.DMA,
    ],
)
def _mixup_sc(x_hbm, out_hbm, *scratch):
    _sc_body(x_hbm, out_hbm, *scratch)


def kernel(x):
    return _mixup_sc(x)


# speedup vs baseline: 5.5072x; 3.2765x over previous
import functools
import numpy as np
import jax
from jax import lax
import jax.numpy as jnp
from jax.experimental import pallas as pl
from jax.experimental.pallas import tpu as pltpu
from jax.experimental.pallas import tpu_sc as plsc

_B = 128
_CC, _HH, _WW = 3, 224, 224
_NPOS = _CC * _HH * _WW
_ALPHA = 0.9
_BETA = 1.0 - _ALPHA

_NC, _NS, _L = 2, 16, 16
_NW = _NC * _NS
_PPW = _NPOS // _NW
_P = 168
_STEPS = _PPW // _P
_NG = _B // _L


_PERM = np.asarray([
    98, 105, 103, 43, 22, 94, 86, 125, 49, 0, 45, 108, 56, 121, 62, 109,
    3, 77, 9, 64, 5, 52, 50, 37, 78, 95, 30, 117, 127, 71, 53, 34,
    83, 18, 14, 116, 46, 1, 74, 124, 58, 92, 51, 81, 107, 48, 100, 42,
    106, 8, 69, 101, 90, 110, 66, 65, 21, 17, 67, 4, 32, 102, 27, 33,
    75, 89, 70, 123, 63, 104, 13, 39, 73, 85, 79, 120, 91, 41, 115, 6,
    59, 2, 57, 35, 99, 19, 40, 72, 118, 54, 80, 31, 126, 26, 97, 36,
    38, 25, 47, 61, 96, 15, 28, 68, 60, 82, 112, 55, 44, 119, 11, 114,
    10, 122, 76, 93, 84, 87, 16, 12, 88, 23, 29, 24, 7, 113, 111, 20,
], dtype=np.int32)

_PTAB = _PERM.reshape(_NG, _L)


def _axpy_chunk(ptab_v, in_ref, o_ref):
    pvecs = [ptab_v[k] for k in range(_NG)]

    @plsc.parallel_loop(0, _P)
    def it(n):
        rown = jnp.full((_L,), n, jnp.int32)
        for k in range(_NG):
            sl = pl.ds(k * _L, _L)
            direct = in_ref[n, sl]
            mixed = plsc.load_gather(in_ref, [rown, pvecs[k]])
            o_ref[n, sl] = _ALPHA * direct + _BETA * mixed


def _sc_body(x_hbm, ptab_hbm, out_hbm, ptab_v,
             i0, i1, o0, o1, si0, si1, so0, so1):
    wid = lax.axis_index("s") * _NC + lax.axis_index("c")
    base = wid * _PPW
    ibufs, obufs = (i0, i1), (o0, o1)
    sis, sos = (si0, si1), (so0, so1)

    pltpu.sync_copy(ptab_hbm, ptab_v)

    def src(s):
        return x_hbm.at[pl.ds(base + _P * s, _P)]

    def dst(s):
        return out_hbm.at[pl.ds(base + _P * s, _P)]

    for j in range(2):
        pltpu.make_async_copy(src(j), ibufs[j], sis[j]).start()

    def step(g, carry):
        for j in range(2):
            s = g * 2 + j
            pltpu.make_async_copy(src(s), ibufs[j], sis[j]).wait()

            @pl.when(s >= 2)
            def _():
                pltpu.make_async_copy(obufs[j], dst(s - 2), sos[j]).wait()

            _axpy_chunk(ptab_v, ibufs[j], obufs[j])
            pltpu.make_async_copy(obufs[j], dst(s), sos[j]).start()

            @pl.when(s < _STEPS - 2)
            def _():
                pltpu.make_async_copy(src(s + 2), ibufs[j], sis[j]).start()
        return carry

    lax.fori_loop(0, _STEPS // 2, step, 0)

    for j in range(2):
        pltpu.make_async_copy(obufs[j], dst(_STEPS - 2 + j), sos[j]).wait()


@functools.partial(
    pl.kernel,
    out_type=jax.ShapeDtypeStruct((_NPOS, _B), jnp.float32),
    mesh=plsc.VectorSubcoreMesh(core_axis_name="c", subcore_axis_name="s"),
    compiler_params=pltpu.CompilerParams(needs_layout_passes=False),
    scratch_types=[
        pltpu.VMEM((_NG, _L), jnp.int32),
        pltpu.VMEM((_P, _B), jnp.float32),
        pltpu.VMEM((_P, _B), jnp.float32),
        pltpu.VMEM((_P, _B), jnp.float32),
        pltpu.VMEM((_P, _B), jnp.float32),
        pltpu.SemaphoreType.DMA,
        pltpu.SemaphoreType.DMA,
        pltpu.SemaphoreType.DMA,
        pltpu.SemaphoreType.DMA,
    ],
)
def _mixup_sc(x_hbm, ptab_hbm, out_hbm, *scratch):
    _sc_body(x_hbm, ptab_hbm, out_hbm, *scratch)


def kernel(x):
    xt = jnp.transpose(x, (1, 2, 3, 0)).reshape(_NPOS, _B)
    ot = _mixup_sc(xt, jnp.asarray(_PTAB))
    return ot.reshape(_CC, _HH, _WW, _B).transpose(3, 0, 1, 2)


# speedup vs baseline: 5.5365x; 1.0053x over previous
import functools
import numpy as np
import jax
from jax import lax
import jax.numpy as jnp
from jax.experimental import pallas as pl
from jax.experimental.pallas import tpu as pltpu
from jax.experimental.pallas import tpu_sc as plsc

_B = 128
_CC, _HH, _WW = 3, 224, 224
_NPOS = _CC * _HH * _WW
_ALPHA = 0.9
_BETA = 1.0 - _ALPHA

_NC, _NS, _L = 2, 16, 16
_NW = _NC * _NS
_PPW = _NPOS // _NW
_P = 168
_STEPS = _PPW // _P
_NG = _B // _L


_PERM = np.asarray([
    98, 105, 103, 43, 22, 94, 86, 125, 49, 0, 45, 108, 56, 121, 62, 109,
    3, 77, 9, 64, 5, 52, 50, 37, 78, 95, 30, 117, 127, 71, 53, 34,
    83, 18, 14, 116, 46, 1, 74, 124, 58, 92, 51, 81, 107, 48, 100, 42,
    106, 8, 69, 101, 90, 110, 66, 65, 21, 17, 67, 4, 32, 102, 27, 33,
    75, 89, 70, 123, 63, 104, 13, 39, 73, 85, 79, 120, 91, 41, 115, 6,
    59, 2, 57, 35, 99, 19, 40, 72, 118, 54, 80, 31, 126, 26, 97, 36,
    38, 25, 47, 61, 96, 15, 28, 68, 60, 82, 112, 55, 44, 119, 11, 114,
    10, 122, 76, 93, 84, 87, 16, 12, 88, 23, 29, 24, 7, 113, 111, 20,
], dtype=np.int32)

_PTAB = _PERM.reshape(_NG, _L)


def _axpy_chunk(pvecs, in_ref, o_ref):
    @plsc.parallel_loop(0, _P, unroll=2)
    def it(n):
        rown = jnp.full((_L,), n, jnp.int32)
        for k in range(_NG):
            sl = pl.ds(k * _L, _L)
            direct = in_ref[n, sl]
            mixed = plsc.load_gather(in_ref, [rown, pvecs[k]])
            o_ref[n, sl] = _ALPHA * direct + _BETA * mixed


def _sc_body(x_hbm, out_hbm,
             i0, i1, o0, o1, si0, si1, so0, so1):
    wid = lax.axis_index("s") * _NC + lax.axis_index("c")
    base = wid * _PPW
    ibufs, obufs = (i0, i1), (o0, o1)
    sis, sos = (si0, si1), (so0, so1)

    lane = lax.iota(jnp.int32, _L)
    pvecs = []
    for k in range(_NG):
        v = lane * 0
        for l in range(_L):
            v = jnp.where(lane == l, int(_PTAB[k, l]), v)
        pvecs.append(v)

    def src(s):
        return x_hbm.at[pl.ds(base + _P * s, _P)]

    def dst(s):
        return out_hbm.at[pl.ds(base + _P * s, _P)]

    for j in range(2):
        pltpu.make_async_copy(src(j), ibufs[j], sis[j]).start()

    def step(g, carry):
        for j in range(2):
            s = g * 2 + j
            pltpu.make_async_copy(src(s), ibufs[j], sis[j]).wait()

            @pl.when(s >= 2)
            def _():
                pltpu.make_async_copy(obufs[j], dst(s - 2), sos[j]).wait()

            _axpy_chunk(pvecs, ibufs[j], obufs[j])
            pltpu.make_async_copy(obufs[j], dst(s), sos[j]).start()

            @pl.when(s < _STEPS - 2)
            def _():
                pltpu.make_async_copy(src(s + 2), ibufs[j], sis[j]).start()
        return carry

    lax.fori_loop(0, _STEPS // 2, step, 0)

    for j in range(2):
        pltpu.make_async_copy(obufs[j], dst(_STEPS - 2 + j), sos[j]).wait()


@functools.partial(
    pl.kernel,
    out_type=jax.ShapeDtypeStruct((_NPOS, _B), jnp.float32),
    mesh=plsc.VectorSubcoreMesh(core_axis_name="c", subcore_axis_name="s"),
    compiler_params=pltpu.CompilerParams(needs_layout_passes=False),
    scratch_types=[
        pltpu.VMEM((_P, _B), jnp.float32),
        pltpu.VMEM((_P, _B), jnp.float32),
        pltpu.VMEM((_P, _B), jnp.float32),
        pltpu.VMEM((_P, _B), jnp.float32),
        pltpu.SemaphoreType.DMA,
        pltpu.SemaphoreType.DMA,
        pltpu.SemaphoreType.DMA,
        pltpu.SemaphoreType.DMA,
    ],
)
def _mixup_sc(x_hbm, out_hbm, *scratch):
    _sc_body(x_hbm, out_hbm, *scratch)


def kernel(x):
    xt = jnp.transpose(x, (1, 2, 3, 0)).reshape(_NPOS, _B)
    ot = _mixup_sc(xt)
    return ot.reshape(_CC, _HH, _WW, _B).transpose(3, 0, 1, 2)
